# Initial kernel scaffold; baseline (speedup 1.0000x reference)
import jax
import jax.numpy as jnp
from jax.experimental import pallas as pl


def kernel(cart, ef, numatoms, species, atom_index, shifts, rs, inta, params, ef_para, hyper):
    raise NotImplementedError("write your pallas kernel here")


# speedup vs baseline: 68.1884x; 68.1884x over previous
import functools

import jax
import jax.numpy as jnp
import numpy as np
from jax.experimental import pallas as pl
from jax.experimental.pallas import tpu as pltpu

_NTYPE = 4
_NWAVE = 8
_NIPSIN = 3
_NANG = 13
_NORBIT = 32
_CUTOFF = 5.0
_AP = 1024
_P = 1280

_INDEX_PARA = (0, 1, 1, 1, 2, 2, 2, 2, 2, 2, 2, 2, 2)


def _body(cart_ref, idx_ref, shifts_ref, sp_ref, ef_ref, rs_ref, inta_ref,
          par_ref, efp_ref, hyp_ref, out_ref, acc_ref, tab_ref):
    j = pl.program_id(1)
    nblk = pl.num_programs(1)

    @pl.when(j == 0)
    def _init():
        acc_ref[...] = jnp.zeros_like(acc_ref)
        sp = sp_ref[0]
        sp_oh = (jax.lax.broadcasted_iota(jnp.int32, (_NTYPE, _AP), 0)
                 == sp).astype(jnp.float32)
        tabs = jnp.concatenate([rs_ref[...], inta_ref[...], par_ref[...]],
                               axis=0)
        tab_ref[...] = jax.lax.dot_general(
            tabs, sp_oh, (((1,), (0,)), ((), ())),
            preferred_element_type=jnp.float32)

    ids = idx_ref[0]
    idx0 = ids[0:1]
    idx1 = ids[1:2]
    iota_a = jax.lax.broadcasted_iota(jnp.int32, (_AP, _P), 0)
    oh0 = (iota_a == idx0).astype(jnp.float32)
    oh1 = (iota_a == idx1).astype(jnp.float32)

    cart_t = cart_ref[0]
    d_raw = jax.lax.dot_general(cart_t, oh1 - oh0, (((1,), (0,)), ((), ())),
                                preferred_element_type=jnp.float32)
    g = jax.lax.dot_general(tab_ref[...], oh1, (((1,), (0,)), ((), ())),
                            preferred_element_type=jnp.float32)

    dvec = d_raw[0:3] + shifts_ref[0]
    d2 = jnp.sum(dvec * dvec, axis=0, keepdims=True)
    d = jnp.sqrt(d2)
    inv_d = 1.0 / d
    c = 0.5 * jnp.cos(d * (np.pi / _CUTOFF)) + 0.5
    dcut = c * c
    rs_a, inta_a, par_a = g[0:8], g[8:16], g[16:24]
    dr = d - rs_a
    rw = jnp.exp(inta_a * dr * dr) * par_a
    u = dvec * inv_d

    angs = [dcut]
    for a in range(3):
        angs.append(dcut * u[a:a + 1])
    for a in range(3):
        for b in range(3):
            angs.append(angs[1 + a] * u[b:b + 1])
    w_t = jnp.concatenate([rw * ang for ang in angs]
                          + [jnp.zeros((128 - _NANG * _NWAVE, _P),
                                       jnp.float32)], axis=0)
    acc_ref[...] += jax.lax.dot_general(
        w_t, oh0, (((1,), (1,)), ((), ())),
        preferred_element_type=jnp.float32)

    @pl.when(j == nblk - 1)
    def _finish():
        e = [ef_ref[0, 0, 0], ef_ref[0, 0, 1], ef_ref[0, 0, 2]]
        ef_ang = [1.0] + e + [e[a] * e[b] for a in range(3) for b in range(3)]
        efp = efp_ref[...]
        dens = jnp.zeros((_AP, _NORBIT), jnp.float32)
        for jj in range(_NANG):
            eo = acc_ref[jj * 8:(jj + 1) * 8, :] + efp * ef_ang[jj]
            h = hyp_ref[_INDEX_PARA[jj]]
            hw = jax.lax.dot_general(eo, h, (((0,), (0,)), ((), ())),
                                     preferred_element_type=jnp.float32)
            dens = dens + hw * hw
        out_ref[...] = dens[:1000, :]


@jax.jit
def kernel(cart, ef, numatoms, species, atom_index, shifts, rs, inta, params,
           ef_para, hyper):
    del numatoms
    nbatch, numatom, _ = cart.shape
    npair = atom_index.shape[2]
    nblk = npair // _P

    cart_t = jnp.zeros((nbatch, 8, _AP), jnp.float32)
    cart_t = cart_t.at[:, 0:3, :numatom].set(cart.transpose(0, 2, 1))
    idx_t = atom_index.transpose(1, 0, 2).astype(jnp.int32)
    shifts_t = shifts.transpose(0, 2, 1)
    sp_p = jnp.zeros((nbatch, 1, _AP), jnp.int32)
    sp_p = sp_p.at[:, 0, :numatom].set(
        species.reshape(nbatch, numatom).astype(jnp.int32))
    ef_r = ef.reshape(nbatch, 1, 3)
    rs_t, inta_t, par_t = rs.T, inta.T, params.T
    efp_c = ef_para.reshape(_NWAVE, 1)

    grid = (nbatch, nblk)
    out = pl.pallas_call(
        _body,
        grid=grid,
        in_specs=[
            pl.BlockSpec((1, 8, _AP), lambda b, j: (b, 0, 0)),
            pl.BlockSpec((1, 2, _P), lambda b, j: (b, 0, j)),
            pl.BlockSpec((1, 3, _P), lambda b, j: (b, 0, j)),
            pl.BlockSpec((1, 1, _AP), lambda b, j: (b, 0, 0)),
            pl.BlockSpec((1, 1, 3), lambda b, j: (b, 0, 0),
                         memory_space=pltpu.SMEM),
            pl.BlockSpec((8, 4), lambda b, j: (0, 0)),
            pl.BlockSpec((8, 4), lambda b, j: (0, 0)),
            pl.BlockSpec((8, 4), lambda b, j: (0, 0)),
            pl.BlockSpec((8, 1), lambda b, j: (0, 0)),
            pl.BlockSpec((3, 8, 32), lambda b, j: (0, 0, 0)),
        ],
        out_specs=pl.BlockSpec((numatom, _NORBIT), lambda b, j: (b, 0)),
        out_shape=jax.ShapeDtypeStruct((nbatch * numatom, _NORBIT),
                                       jnp.float32),
        scratch_shapes=[
            pltpu.VMEM((128, _AP), jnp.float32),
            pltpu.VMEM((24, _AP), jnp.float32),
        ],
        compiler_params=pltpu.CompilerParams(
            dimension_semantics=("arbitrary", "arbitrary")),
    )(cart_t, idx_t, shifts_t, sp_p, ef_r, rs_t, inta_t, par_t, efp_c, hyper)
    return out
